# baseline (speedup 1.0000x reference)
import math

import jax
import jax.numpy as jnp
from jax import lax
from jax.experimental import pallas as pl
from jax.experimental.pallas import tpu as pltpu
from jax.experimental.pallas import tpu_sc as plsc

D_MODEL = 768
HALF = D_MODEL // 2
B, N = 64, 1024
TOTAL = B * N
GRID = int(math.sqrt(N)) + 1
NVAL = GRID + 1
CSTRIDE = 64
TPAD = HALF + 1

IDX_SUB, IDX_LANE = 512, 128

NW = 32
ROWS_PER_W = TOTAL // NW
CHUNK = 64
CHUNKS_PER_W = ROWS_PER_W // CHUNK


def _tc_body(xs_ref, ys_ref, row_ref, col_ref, idx_ref, rt_ref, ct_ref):
    xs = xs_ref[...]
    ys = ys_ref[...]
    m = jnp.maximum(jnp.max(xs), jnp.max(ys))
    r = jnp.clip(((ys / m) * float(GRID)).astype(jnp.int32), 0, NVAL - 1)
    c = jnp.clip(((xs / m) * float(GRID)).astype(jnp.int32), 0, NVAL - 1)
    idx_ref[...] = r * CSTRIDE + c
    rt_ref[:, :HALF] = row_ref[...]
    ct_ref[:, :HALF] = col_ref[...]
    rt_ref[:, HALF:] = jnp.zeros((NVAL, 1), jnp.float32)
    ct_ref[:, HALF:] = jnp.zeros((NVAL, 1), jnp.float32)


def _tc_index_and_tables(xs, ys, row34, col34):
    return pl.pallas_call(
        _tc_body,
        out_shape=(
            jax.ShapeDtypeStruct((IDX_SUB, IDX_LANE), jnp.int32),
            jax.ShapeDtypeStruct((NVAL, TPAD), jnp.float32),
            jax.ShapeDtypeStruct((NVAL, TPAD), jnp.float32),
        ),
    )(xs, ys, row34, col34)


def _sc_body(rt_hbm, ct_hbm, idx_hbm, out_hbm,
             idx_vf, rt, ct, bc0, bc1, psem, sc0, sc1):
    wid = lax.axis_index("s") * 2 + lax.axis_index("c")
    cp0 = pltpu.async_copy(
        idx_hbm.at[pl.ds(wid * ROWS_PER_W, ROWS_PER_W)], idx_vf, psem)
    cp1 = pltpu.async_copy(rt_hbm, rt, psem)
    cp2 = pltpu.async_copy(ct_hbm, ct, psem)
    out_base = wid * ROWS_PER_W

    def wstart(k, buf, sem):
        return pltpu.async_copy(
            buf, out_hbm.at[pl.ds(out_base + k * CHUNK, CHUNK)], sem)

    def wwait(buf, sem):
        pltpu.make_async_copy(
            buf, out_hbm.at[pl.ds(out_base, CHUNK)], sem).wait()

    lane_dn = lax.GatherDimensionNumbers(
        offset_dims=(), collapsed_slice_dims=(0,), start_index_map=(0,))

    def lane_splat(x, l):
        return lax.gather(x, jnp.full((16, 1), l, jnp.int32), lane_dn, (1,),
                          mode=lax.GatherScatterMode.PROMISE_IN_BOUNDS)

    def construct(j, buf):
        base = j * CHUNK
        for g in range(CHUNK // 16):
            iv = idx_vf[pl.ds(base + 16 * g, 16)]
            rvec = lax.shift_right_logical(iv, 6) * TPAD
            cvec = jnp.bitwise_and(iv, CSTRIDE - 1) * TPAD

            @plsc.parallel_loop(0, 16, unroll=2)
            def _row(l):
                rs = lane_splat(rvec, l)
                cs = lane_splat(cvec, l)
                row = 16 * g + l

                @plsc.parallel_loop(0, HALF, step=16, unroll=8)
                def _col(k):
                    kv = lax.broadcasted_iota(jnp.int32, (16,), 0) + k
                    buf[row, pl.ds(k, 16)] = plsc.load_gather(rt, [rs + kv])
                    buf[row, pl.ds(HALF + k, 16)] = (
                        plsc.load_gather(ct, [cs + kv]))

    cp0.wait()
    cp1.wait()
    cp2.wait()

    wstart(0, bc0, sc0)
    wstart(1, bc1, sc1)

    def macro(t, carry):
        wwait(bc0, sc0)
        construct(2 * t, bc0)
        wstart(2 * t, bc0, sc0)
        wwait(bc1, sc1)
        construct(2 * t + 1, bc1)
        wstart(2 * t + 1, bc1, sc1)
        return carry

    lax.fori_loop(0, CHUNKS_PER_W // 2, macro, 0, unroll=False)

    wwait(bc0, sc0)
    wwait(bc1, sc1)


def _sc_construct(rt, ct, idx_flat):
    mesh = plsc.VectorSubcoreMesh(core_axis_name="c", subcore_axis_name="s")
    return pl.kernel(
        _sc_body,
        mesh=mesh,
        compiler_params=pltpu.CompilerParams(needs_layout_passes=False),
        out_type=jax.ShapeDtypeStruct((TOTAL, D_MODEL), jnp.float32),
        scratch_types=[
            pltpu.VMEM((ROWS_PER_W,), jnp.int32),
            pltpu.VMEM((NVAL * TPAD,), jnp.float32),
            pltpu.VMEM((NVAL * TPAD,), jnp.float32),
            pltpu.VMEM((CHUNK, D_MODEL), jnp.float32),
            pltpu.VMEM((CHUNK, D_MODEL), jnp.float32),
            pltpu.SemaphoreType.DMA,
            pltpu.SemaphoreType.DMA,
            pltpu.SemaphoreType.DMA,
        ],
    )(rt, ct, idx_flat)


def kernel(patch_coords, row_embed, col_embed):
    xs = patch_coords[:, :, 0].reshape(IDX_SUB, IDX_LANE)
    ys = patch_coords[:, :, 1].reshape(IDX_SUB, IDX_LANE)
    row34 = row_embed[:NVAL]
    col34 = col_embed[:NVAL]
    idx, rt, ct = _tc_index_and_tables(xs, ys, row34, col34)
    out = _sc_construct(rt.reshape(NVAL * TPAD), ct.reshape(NVAL * TPAD),
                        idx.reshape(TOTAL))
    return out.reshape(B, N, D_MODEL)


# speedup vs baseline: 1.0487x; 1.0487x over previous
import math

import jax
import jax.numpy as jnp
from jax import lax
from jax.experimental import pallas as pl
from jax.experimental.pallas import tpu as pltpu
from jax.experimental.pallas import tpu_sc as plsc

D_MODEL = 768
HALF = D_MODEL // 2
B, N = 64, 1024
TOTAL = B * N
GRID = int(math.sqrt(N)) + 1
NVAL = GRID + 1
CSTRIDE = 64
TPAD = HALF + 1

IDX_SUB, IDX_LANE = 512, 128

NW = 32
ROWS_PER_W = TOTAL // NW
CHUNK = 32
CHUNKS_PER_W = ROWS_PER_W // CHUNK


def _tc_body(xs_ref, ys_ref, row_ref, col_ref, idx_ref, rt_ref, ct_ref):
    xs = xs_ref[...]
    ys = ys_ref[...]
    m = jnp.maximum(jnp.max(xs), jnp.max(ys))
    r = jnp.clip(((ys / m) * float(GRID)).astype(jnp.int32), 0, NVAL - 1)
    c = jnp.clip(((xs / m) * float(GRID)).astype(jnp.int32), 0, NVAL - 1)
    idx_ref[...] = r * CSTRIDE + c
    rt_ref[:, :HALF] = row_ref[...]
    ct_ref[:, :HALF] = col_ref[...]
    rt_ref[:, HALF:] = jnp.zeros((NVAL, 1), jnp.float32)
    ct_ref[:, HALF:] = jnp.zeros((NVAL, 1), jnp.float32)


def _tc_index_and_tables(xs, ys, row34, col34):
    return pl.pallas_call(
        _tc_body,
        out_shape=(
            jax.ShapeDtypeStruct((IDX_SUB, IDX_LANE), jnp.int32),
            jax.ShapeDtypeStruct((NVAL, TPAD), jnp.float32),
            jax.ShapeDtypeStruct((NVAL, TPAD), jnp.float32),
        ),
    )(xs, ys, row34, col34)


def _sc_body(rt_hbm, ct_hbm, idx_hbm, out_hbm,
             idx_vf, rt, ct, bc0, bc1, psem, sc0, sc1):
    wid = lax.axis_index("s") * 2 + lax.axis_index("c")
    cp0 = pltpu.async_copy(
        idx_hbm.at[pl.ds(wid * ROWS_PER_W, ROWS_PER_W)], idx_vf, psem)
    cp1 = pltpu.async_copy(rt_hbm, rt, psem)
    cp2 = pltpu.async_copy(ct_hbm, ct, psem)
    out_base = wid * ROWS_PER_W

    def wstart(k, buf, sem):
        return pltpu.async_copy(
            buf, out_hbm.at[pl.ds(out_base + k * CHUNK, CHUNK)], sem)

    def wwait(buf, sem):
        pltpu.make_async_copy(
            buf, out_hbm.at[pl.ds(out_base, CHUNK)], sem).wait()

    lane_dn = lax.GatherDimensionNumbers(
        offset_dims=(), collapsed_slice_dims=(0,), start_index_map=(0,))

    def lane_splat(x, l):
        return lax.gather(x, jnp.full((16, 1), l, jnp.int32), lane_dn, (1,),
                          mode=lax.GatherScatterMode.PROMISE_IN_BOUNDS)

    def construct(j, buf):
        base = j * CHUNK
        for g in range(CHUNK // 16):
            iv = idx_vf[pl.ds(base + 16 * g, 16)]
            rvec = lax.shift_right_logical(iv, 6) * TPAD
            cvec = jnp.bitwise_and(iv, CSTRIDE - 1) * TPAD

            @plsc.parallel_loop(0, 16, unroll=4)
            def _row(l):
                rs = lane_splat(rvec, l)
                cs = lane_splat(cvec, l)
                row = 16 * g + l

                @plsc.parallel_loop(0, HALF, step=16, unroll=8)
                def _col(k):
                    kv = lax.broadcasted_iota(jnp.int32, (16,), 0) + k
                    buf[row, pl.ds(k, 16)] = plsc.load_gather(rt, [rs + kv])
                    buf[row, pl.ds(HALF + k, 16)] = (
                        plsc.load_gather(ct, [cs + kv]))

    cp0.wait()
    cp1.wait()
    cp2.wait()

    wstart(0, bc0, sc0)
    wstart(1, bc1, sc1)

    def macro(t, carry):
        wwait(bc0, sc0)
        construct(2 * t, bc0)
        wstart(2 * t, bc0, sc0)
        wwait(bc1, sc1)
        construct(2 * t + 1, bc1)
        wstart(2 * t + 1, bc1, sc1)
        return carry

    lax.fori_loop(0, CHUNKS_PER_W // 2, macro, 0, unroll=False)

    wwait(bc0, sc0)
    wwait(bc1, sc1)


def _sc_construct(rt, ct, idx_flat):
    mesh = plsc.VectorSubcoreMesh(core_axis_name="c", subcore_axis_name="s")
    return pl.kernel(
        _sc_body,
        mesh=mesh,
        compiler_params=pltpu.CompilerParams(needs_layout_passes=False),
        out_type=jax.ShapeDtypeStruct((TOTAL, D_MODEL), jnp.float32),
        scratch_types=[
            pltpu.VMEM((ROWS_PER_W,), jnp.int32),
            pltpu.VMEM((NVAL * TPAD,), jnp.float32),
            pltpu.VMEM((NVAL * TPAD,), jnp.float32),
            pltpu.VMEM((CHUNK, D_MODEL), jnp.float32),
            pltpu.VMEM((CHUNK, D_MODEL), jnp.float32),
            pltpu.SemaphoreType.DMA,
            pltpu.SemaphoreType.DMA,
            pltpu.SemaphoreType.DMA,
        ],
    )(rt, ct, idx_flat)


def kernel(patch_coords, row_embed, col_embed):
    xs = patch_coords[:, :, 0].reshape(IDX_SUB, IDX_LANE)
    ys = patch_coords[:, :, 1].reshape(IDX_SUB, IDX_LANE)
    row34 = row_embed[:NVAL]
    col34 = col_embed[:NVAL]
    idx, rt, ct = _tc_index_and_tables(xs, ys, row34, col34)
    out = _sc_construct(rt.reshape(NVAL * TPAD), ct.reshape(NVAL * TPAD),
                        idx.reshape(TOTAL))
    return out.reshape(B, N, D_MODEL)
